# baseline (speedup 1.0000x reference)
import jax
import jax.numpy as jnp
from jax.experimental import pallas as pl
from jax.experimental.pallas import tpu as pltpu

_TI = 256


def _sage_kernel(adj_ref, x_ref, mask_ref, w1t_ref, b1_ref, w2t_ref, b2_ref,
                 wfct_ref, bfc_ref, out_ref,
                 a_buf, xp0_buf, h1_buf, xp1_buf, inv_buf, pooled_buf):
    b = pl.program_id(0)
    p = pl.program_id(1)
    i = pl.program_id(2)
    nb = pl.num_programs(0)
    ni = pl.num_programs(2)
    r0 = i * _TI

    m = mask_ref[0]

    @pl.when(p == 0)
    def _layer1():
        @pl.when(i == 0)
        def _project_x():
            xp0_buf[...] = jnp.dot(x_ref[0], w1t_ref[...],
                                   preferred_element_type=jnp.float32)

        af = (adj_ref[0] > 0.98).astype(jnp.float32)
        deg = jnp.sum(af, axis=1, keepdims=True)
        inv = 1.0 / (deg + 1.0)
        inv_buf[pl.ds(r0, _TI), :] = inv
        a_buf[pl.ds(r0, _TI), :] = af.astype(jnp.bfloat16)
        agg = jnp.dot(af, xp0_buf[...],
                      preferred_element_type=jnp.float32)
        xp0_t = xp0_buf[pl.ds(r0, _TI), :]
        h1 = jnp.maximum((agg + xp0_t) * inv + b1_ref[...], 0.0) * m
        h1_buf[pl.ds(r0, _TI), :] = h1

    @pl.when(p == 1)
    def _layer2():
        @pl.when(i == 0)
        def _project_h1():
            xp1_buf[...] = jnp.dot(h1_buf[...], w2t_ref[...],
                                   preferred_element_type=jnp.float32)

        af = a_buf[pl.ds(r0, _TI), :].astype(jnp.float32)
        agg = jnp.dot(af, xp1_buf[...],
                      preferred_element_type=jnp.float32)
        xp1_t = xp1_buf[pl.ds(r0, _TI), :]
        inv = inv_buf[pl.ds(r0, _TI), :]
        h2 = jnp.maximum((agg + xp1_t) * inv + b2_ref[...], 0.0) * m
        tmax = jnp.max(h2, axis=0, keepdims=True)

        @pl.when(i == 0)
        def _pool_init():
            pooled_buf[pl.ds(b, 1), :] = tmax

        @pl.when(i > 0)
        def _pool_acc():
            pooled_buf[pl.ds(b, 1), :] = jnp.maximum(
                pooled_buf[pl.ds(b, 1), :], tmax)

        @pl.when((b == nb - 1) & (i == ni - 1))
        def _final_fc():
            out_ref[...] = jnp.dot(pooled_buf[...], wfct_ref[...],
                                   preferred_element_type=jnp.float32) \
                + bfc_ref[...]


def kernel(x, adj, mask, W1, b1, W2, b2, Wfc, bfc):
    B, N, F = x.shape
    H1 = W1.shape[0]
    H2 = W2.shape[0]
    OUT = Wfc.shape[0]
    ni = N // _TI

    mask3 = mask.reshape(B, N, 1)
    grid = (B, 2, ni)

    out = pl.pallas_call(
        _sage_kernel,
        grid=grid,
        in_specs=[
            pl.BlockSpec((1, _TI, N),
                         lambda b, p, i: (b, jnp.where(p == 0, i, ni - 1), 0)),
            pl.BlockSpec((1, N, F), lambda b, p, i: (b, 0, 0)),
            pl.BlockSpec((1, _TI, 1), lambda b, p, i: (b, i, 0)),
            pl.BlockSpec((F, H1), lambda b, p, i: (0, 0)),
            pl.BlockSpec((1, H1), lambda b, p, i: (0, 0)),
            pl.BlockSpec((H1, H2), lambda b, p, i: (0, 0)),
            pl.BlockSpec((1, H2), lambda b, p, i: (0, 0)),
            pl.BlockSpec((H2, OUT), lambda b, p, i: (0, 0)),
            pl.BlockSpec((1, OUT), lambda b, p, i: (0, 0)),
        ],
        out_specs=pl.BlockSpec((B, OUT), lambda b, p, i: (0, 0)),
        out_shape=jax.ShapeDtypeStruct((B, OUT), jnp.float32),
        scratch_shapes=[
            pltpu.VMEM((N, N), jnp.bfloat16),
            pltpu.VMEM((N, H1), jnp.float32),
            pltpu.VMEM((N, H1), jnp.float32),
            pltpu.VMEM((N, H2), jnp.float32),
            pltpu.VMEM((N, 1), jnp.float32),
            pltpu.VMEM((B, H2), jnp.float32),
        ],
    )(adj, x, mask3, W1.T, b1.reshape(1, H1), W2.T, b2.reshape(1, H2),
      Wfc.T, bfc.reshape(1, OUT))
    return out


# speedup vs baseline: 1.0098x; 1.0098x over previous
import jax
import jax.numpy as jnp
from jax.experimental import pallas as pl
from jax.experimental.pallas import tpu as pltpu

_TI = 256


def _sage_kernel(adj_ref, x_ref, mask_ref, w1t_ref, b1_ref, w2t_ref, b2_ref,
                 wfct_ref, bfc_ref, out_ref,
                 a_buf, xp0_buf, h1_buf, xp1_buf, inv_buf, pooled_buf):
    b = pl.program_id(0)
    p = pl.program_id(1)
    i = pl.program_id(2)
    nb = pl.num_programs(0)
    ni = pl.num_programs(2)
    r0 = i * _TI

    m = mask_ref[0]

    @pl.when(p == 0)
    def _layer1():
        @pl.when(i == 0)
        def _project_x():
            xp0_buf[...] = jnp.dot(x_ref[0], w1t_ref[...],
                                   preferred_element_type=jnp.float32)

        af = (adj_ref[0] > 0.98).astype(jnp.bfloat16)
        deg = jnp.sum(af.astype(jnp.float32), axis=1, keepdims=True)
        inv = 1.0 / (deg + 1.0)
        inv_buf[pl.ds(r0, _TI), :] = inv
        a_buf[pl.ds(r0, _TI), :] = af
        agg = jnp.dot(af, xp0_buf[...].astype(jnp.bfloat16),
                      preferred_element_type=jnp.float32)
        xp0_t = xp0_buf[pl.ds(r0, _TI), :]
        h1 = jnp.maximum((agg + xp0_t) * inv + b1_ref[...], 0.0) * m
        h1_buf[pl.ds(r0, _TI), :] = h1

    @pl.when(p == 1)
    def _layer2():
        @pl.when(i == 0)
        def _project_h1():
            xp1_buf[...] = jnp.dot(h1_buf[...], w2t_ref[...],
                                   preferred_element_type=jnp.float32)

        af = a_buf[pl.ds(r0, _TI), :]
        agg = jnp.dot(af, xp1_buf[...].astype(jnp.bfloat16),
                      preferred_element_type=jnp.float32)
        xp1_t = xp1_buf[pl.ds(r0, _TI), :]
        inv = inv_buf[pl.ds(r0, _TI), :]
        h2 = jnp.maximum((agg + xp1_t) * inv + b2_ref[...], 0.0) * m
        tmax = jnp.max(h2, axis=0, keepdims=True)

        @pl.when(i == 0)
        def _pool_init():
            pooled_buf[pl.ds(b, 1), :] = tmax

        @pl.when(i > 0)
        def _pool_acc():
            pooled_buf[pl.ds(b, 1), :] = jnp.maximum(
                pooled_buf[pl.ds(b, 1), :], tmax)

        @pl.when((b == nb - 1) & (i == ni - 1))
        def _final_fc():
            out_ref[...] = jnp.dot(pooled_buf[...], wfct_ref[...],
                                   preferred_element_type=jnp.float32) \
                + bfc_ref[...]


def kernel(x, adj, mask, W1, b1, W2, b2, Wfc, bfc):
    B, N, F = x.shape
    H1 = W1.shape[0]
    H2 = W2.shape[0]
    OUT = Wfc.shape[0]
    ni = N // _TI

    mask3 = mask.reshape(B, N, 1)
    grid = (B, 2, ni)

    out = pl.pallas_call(
        _sage_kernel,
        grid=grid,
        in_specs=[
            pl.BlockSpec((1, _TI, N),
                         lambda b, p, i: (b, jnp.where(p == 0, i, ni - 1), 0)),
            pl.BlockSpec((1, N, F), lambda b, p, i: (b, 0, 0)),
            pl.BlockSpec((1, _TI, 1), lambda b, p, i: (b, i, 0)),
            pl.BlockSpec((F, H1), lambda b, p, i: (0, 0)),
            pl.BlockSpec((1, H1), lambda b, p, i: (0, 0)),
            pl.BlockSpec((H1, H2), lambda b, p, i: (0, 0)),
            pl.BlockSpec((1, H2), lambda b, p, i: (0, 0)),
            pl.BlockSpec((H2, OUT), lambda b, p, i: (0, 0)),
            pl.BlockSpec((1, OUT), lambda b, p, i: (0, 0)),
        ],
        out_specs=pl.BlockSpec((B, OUT), lambda b, p, i: (0, 0)),
        out_shape=jax.ShapeDtypeStruct((B, OUT), jnp.float32),
        scratch_shapes=[
            pltpu.VMEM((N, N), jnp.bfloat16),
            pltpu.VMEM((N, H1), jnp.float32),
            pltpu.VMEM((N, H1), jnp.float32),
            pltpu.VMEM((N, H2), jnp.float32),
            pltpu.VMEM((N, 1), jnp.float32),
            pltpu.VMEM((B, H2), jnp.float32),
        ],
    )(adj, x, mask3, W1.T, b1.reshape(1, H1), W2.T, b2.reshape(1, H2),
      Wfc.T, bfc.reshape(1, OUT))
    return out


# speedup vs baseline: 1.1379x; 1.1268x over previous
import jax
import jax.numpy as jnp
from jax.experimental import pallas as pl
from jax.experimental.pallas import tpu as pltpu

_TI = 256

_NT = (((1,), (1,)), ((), ()))


def _sage_kernel(adj_ref, x_ref, mask_ref, w1_ref, b1_ref, w2_ref, b2_ref,
                 wfc_ref, bfc_ref, out_ref,
                 a_buf, xp0t_buf, h1t_buf, xp1t_buf, macc_buf):
    b = pl.program_id(0)
    p = pl.program_id(1)
    i = pl.program_id(2)
    nb = pl.num_programs(0)
    ni = pl.num_programs(2)
    r0 = i * _TI

    m = mask_ref[0]

    @pl.when(p == 0)
    def _layer1():
        @pl.when(i == 0)
        def _project_x():
            xp0t = jax.lax.dot_general(w1_ref[...], x_ref[0], _NT,
                                       preferred_element_type=jnp.float32)
            xp0t_buf[0:64, :] = xp0t.astype(jnp.bfloat16)
            xp0t_buf[64:72, :] = jnp.ones((8, xp0t.shape[1]), jnp.bfloat16)

        af = (adj_ref[0] > 0.98).astype(jnp.bfloat16)
        a_buf[pl.ds(r0, _TI), :] = af
        comb = jax.lax.dot_general(xp0t_buf[...], af, _NT,
                                   preferred_element_type=jnp.float32)
        comb = comb + xp0t_buf[:, pl.ds(r0, _TI)].astype(jnp.float32)
        inv = 1.0 / comb[64:65, :]
        h1 = jnp.maximum(comb[0:64, :] * inv + b1_ref[...], 0.0) * m
        h1t_buf[:, pl.ds(r0, _TI)] = h1

    @pl.when(p == 1)
    def _layer2():
        @pl.when(i == 0)
        def _project_h1():
            xp1t = jnp.dot(w2_ref[...], h1t_buf[...],
                           preferred_element_type=jnp.float32)
            xp1t_buf[0:32, :] = xp1t.astype(jnp.bfloat16)
            xp1t_buf[32:40, :] = jnp.ones((8, xp1t.shape[1]), jnp.bfloat16)

        af = a_buf[pl.ds(r0, _TI), :]
        comb = jax.lax.dot_general(xp1t_buf[...], af, _NT,
                                   preferred_element_type=jnp.float32)
        comb = comb + xp1t_buf[:, pl.ds(r0, _TI)].astype(jnp.float32)
        inv = 1.0 / comb[32:33, :]
        h2 = jnp.maximum(comb[0:32, :] * inv + b2_ref[...], 0.0) * m

        @pl.when(i == 0)
        def _pool_init():
            macc_buf[...] = h2

        @pl.when(i > 0)
        def _pool_acc():
            macc_buf[...] = jnp.maximum(macc_buf[...], h2)

        @pl.when(i == ni - 1)
        def _final_fc():
            pooled = jnp.max(macc_buf[...], axis=1, keepdims=True)
            outcol = jnp.dot(wfc_ref[...], pooled,
                             preferred_element_type=jnp.float32) + bfc_ref[...]
            out_ref[...] = jnp.broadcast_to(outcol, out_ref.shape)


def kernel(x, adj, mask, W1, b1, W2, b2, Wfc, bfc):
    B, N, F = x.shape
    H1 = W1.shape[0]
    H2 = W2.shape[0]
    OUT = Wfc.shape[0]
    ni = N // _TI
    grid = (B, 2, ni)

    outt = pl.pallas_call(
        _sage_kernel,
        grid=grid,
        in_specs=[
            pl.BlockSpec((1, _TI, N),
                         lambda b, p, i: (b, jnp.where(p == 0, i, ni - 1), 0)),
            pl.BlockSpec((1, N, F), lambda b, p, i: (b, 0, 0)),
            pl.BlockSpec((1, 1, _TI), lambda b, p, i: (b, 0, i)),
            pl.BlockSpec((H1, F), lambda b, p, i: (0, 0)),
            pl.BlockSpec((H1, 1), lambda b, p, i: (0, 0)),
            pl.BlockSpec((H2, H1), lambda b, p, i: (0, 0)),
            pl.BlockSpec((H2, 1), lambda b, p, i: (0, 0)),
            pl.BlockSpec((OUT, H2), lambda b, p, i: (0, 0)),
            pl.BlockSpec((OUT, 1), lambda b, p, i: (0, 0)),
        ],
        out_specs=pl.BlockSpec((OUT, 128), lambda b, p, i: (0, b)),
        out_shape=jax.ShapeDtypeStruct((OUT, B * 128), jnp.float32),
        scratch_shapes=[
            pltpu.VMEM((N, N), jnp.bfloat16),
            pltpu.VMEM((72, N), jnp.bfloat16),
            pltpu.VMEM((H1, N), jnp.float32),
            pltpu.VMEM((40, N), jnp.bfloat16),
            pltpu.VMEM((H2, _TI), jnp.float32),
        ],
    )(adj, x, mask.reshape(B, 1, N), W1, b1.reshape(H1, 1), W2, b2.reshape(H2, 1),
      Wfc, bfc.reshape(OUT, 1))
    return outt.reshape(OUT, B, 128)[:, :, 0].T


# speedup vs baseline: 2.3486x; 2.0639x over previous
import jax
import jax.numpy as jnp
from jax.experimental import pallas as pl
from jax.experimental.pallas import tpu as pltpu

_NT = (((1,), (1,)), ((), ()))


def _sage_kernel(adj_ref, x_ref, mask_ref, w1_ref, b1_ref, w2_ref, b2_ref,
                 wfc_ref, bfc_ref, out_ref,
                 a_buf, xp0t_buf, h1t_buf, xp1t_buf):
    m = mask_ref[0]
    n = m.shape[1]

    xp0t = jax.lax.dot_general(w1_ref[...], x_ref[0], _NT,
                               preferred_element_type=jnp.float32)
    xp0t_buf[0:64, :] = xp0t.astype(jnp.bfloat16)
    xp0t_buf[64:72, :] = jnp.ones((8, n), jnp.bfloat16)

    a_buf[...] = (adj_ref[0] > 0.98).astype(jnp.bfloat16)

    comb = jax.lax.dot_general(xp0t_buf[...], a_buf[...], _NT,
                               preferred_element_type=jnp.float32)
    comb = comb + xp0t_buf[...].astype(jnp.float32)
    inv = 1.0 / comb[64:65, :]
    h1t_buf[...] = jnp.maximum(comb[0:64, :] * inv + b1_ref[...], 0.0) * m

    xp1t = jnp.dot(w2_ref[...], h1t_buf[...],
                   preferred_element_type=jnp.float32)
    xp1t_buf[0:32, :] = xp1t.astype(jnp.bfloat16)
    xp1t_buf[32:40, :] = jnp.ones((8, n), jnp.bfloat16)

    comb2 = jax.lax.dot_general(xp1t_buf[...], a_buf[...], _NT,
                                preferred_element_type=jnp.float32)
    comb2 = comb2 + xp1t_buf[...].astype(jnp.float32)
    inv2 = 1.0 / comb2[32:33, :]
    h2 = jnp.maximum(comb2[0:32, :] * inv2 + b2_ref[...], 0.0) * m

    pooled = jnp.max(h2, axis=1, keepdims=True)
    outcol = jnp.dot(wfc_ref[...], pooled,
                     preferred_element_type=jnp.float32) + bfc_ref[...]
    out_ref[...] = jnp.broadcast_to(outcol, out_ref.shape)


def kernel(x, adj, mask, W1, b1, W2, b2, Wfc, bfc):
    B, N, F = x.shape
    H1 = W1.shape[0]
    H2 = W2.shape[0]
    OUT = Wfc.shape[0]

    outt = pl.pallas_call(
        _sage_kernel,
        grid=(B,),
        in_specs=[
            pl.BlockSpec((1, N, N), lambda b: (b, 0, 0)),
            pl.BlockSpec((1, N, F), lambda b: (b, 0, 0)),
            pl.BlockSpec((1, 1, N), lambda b: (b, 0, 0)),
            pl.BlockSpec((H1, F), lambda b: (0, 0)),
            pl.BlockSpec((H1, 1), lambda b: (0, 0)),
            pl.BlockSpec((H2, H1), lambda b: (0, 0)),
            pl.BlockSpec((H2, 1), lambda b: (0, 0)),
            pl.BlockSpec((OUT, H2), lambda b: (0, 0)),
            pl.BlockSpec((OUT, 1), lambda b: (0, 0)),
        ],
        out_specs=pl.BlockSpec((OUT, 128), lambda b: (0, b)),
        out_shape=jax.ShapeDtypeStruct((OUT, B * 128), jnp.float32),
        scratch_shapes=[
            pltpu.VMEM((N, N), jnp.bfloat16),
            pltpu.VMEM((72, N), jnp.bfloat16),
            pltpu.VMEM((H1, N), jnp.float32),
            pltpu.VMEM((40, N), jnp.bfloat16),
        ],
    )(adj, x, mask.reshape(B, 1, N), W1, b1.reshape(H1, 1), W2,
      b2.reshape(H2, 1), Wfc, bfc.reshape(OUT, 1))
    return outt.reshape(OUT, B, 128)[:, :, 0].T


# speedup vs baseline: 3.1360x; 1.3353x over previous
import jax
import jax.numpy as jnp
from jax.experimental import pallas as pl
from jax.experimental.pallas import tpu as pltpu

_NT = (((1,), (1,)), ((), ()))
_OUTER = (((0,), (0,)), ((), ()))


def _sage_kernel(adj_ref, x_ref, mask_ref, w1_ref, b1_ref, w2_ref, b2_ref,
                 wfc_ref, bfc_ref, out_ref,
                 a_buf, xp0t_buf, h1t_buf, xp1t_buf):
    b = pl.program_id(0)
    m = mask_ref[pl.ds(b, 1), :]
    n = m.shape[1]
    ones_row = jnp.ones((1, n), jnp.float32)

    b1c = jax.lax.dot_general(b1_ref[...], ones_row, _OUTER,
                              preferred_element_type=jnp.float32)
    b2c = jax.lax.dot_general(b2_ref[...], ones_row, _OUTER,
                              preferred_element_type=jnp.float32)

    xp0t = jax.lax.dot_general(w1_ref[...], x_ref[0], _NT,
                               preferred_element_type=jnp.float32)
    xp0t_buf[0:64, :] = xp0t.astype(jnp.bfloat16)
    xp0t_buf[64:72, :] = jnp.ones((8, n), jnp.bfloat16)

    a_buf[...] = (adj_ref[0] > 0.98).astype(jnp.bfloat16)

    comb = jax.lax.dot_general(xp0t_buf[...], a_buf[...], _NT,
                               preferred_element_type=jnp.float32)
    comb = comb + xp0t_buf[...].astype(jnp.float32)
    inv = 1.0 / comb[64:65, :]
    h1t_buf[...] = jnp.maximum(comb[0:64, :] * inv + b1c, 0.0) * m

    xp1t = jnp.dot(w2_ref[...], h1t_buf[...],
                   preferred_element_type=jnp.float32)
    xp1t_buf[0:32, :] = xp1t.astype(jnp.bfloat16)
    xp1t_buf[32:40, :] = jnp.ones((8, n), jnp.bfloat16)

    comb2 = jax.lax.dot_general(xp1t_buf[...], a_buf[...], _NT,
                                preferred_element_type=jnp.float32)
    comb2 = comb2 + xp1t_buf[...].astype(jnp.float32)
    inv2 = 1.0 / comb2[32:33, :]
    h2 = jnp.maximum(comb2[0:32, :] * inv2 + b2c, 0.0) * m

    pooled = jnp.max(h2, axis=1, keepdims=True)
    outrow = jax.lax.dot_general(pooled, wfc_ref[...],
                                 (((0,), (1,)), ((), ())),
                                 preferred_element_type=jnp.float32) \
        + bfc_ref[...]
    out_ref[pl.ds(b, 1), :] = outrow


def kernel(x, adj, mask, W1, b1, W2, b2, Wfc, bfc):
    B, N, F = x.shape
    H1 = W1.shape[0]
    H2 = W2.shape[0]
    OUT = Wfc.shape[0]

    return pl.pallas_call(
        _sage_kernel,
        grid=(B,),
        in_specs=[
            pl.BlockSpec((1, N, N), lambda b: (b, 0, 0)),
            pl.BlockSpec((1, N, F), lambda b: (b, 0, 0)),
            pl.BlockSpec((B, N), lambda b: (0, 0)),
            pl.BlockSpec((H1, F), lambda b: (0, 0)),
            pl.BlockSpec((1, H1), lambda b: (0, 0)),
            pl.BlockSpec((H2, H1), lambda b: (0, 0)),
            pl.BlockSpec((1, H2), lambda b: (0, 0)),
            pl.BlockSpec((OUT, H2), lambda b: (0, 0)),
            pl.BlockSpec((1, OUT), lambda b: (0, 0)),
        ],
        out_specs=pl.BlockSpec((B, OUT), lambda b: (0, 0)),
        out_shape=jax.ShapeDtypeStruct((B, OUT), jnp.float32),
        scratch_shapes=[
            pltpu.VMEM((N, N), jnp.bfloat16),
            pltpu.VMEM((72, N), jnp.bfloat16),
            pltpu.VMEM((H1, N), jnp.float32),
            pltpu.VMEM((40, N), jnp.bfloat16),
        ],
    )(adj, x, mask, W1, b1.reshape(1, H1), W2, b2.reshape(1, H2),
      Wfc, bfc.reshape(1, OUT))


# speedup vs baseline: 3.3847x; 1.0793x over previous
import jax
import jax.numpy as jnp
from jax.experimental import pallas as pl
from jax.experimental.pallas import tpu as pltpu

_NT = (((1,), (1,)), ((), ()))
_OUTER = (((0,), (0,)), ((), ()))
_GPB = 2


def _sage_kernel(adj_ref, x_ref, mask_ref, w1_ref, b1_ref, w2_ref, b2_ref,
                 wfc_ref, bfc_ref, out_ref,
                 a_buf, xp0t_buf, h1t_buf, xp1t_buf):
    s = pl.program_id(0)
    n = adj_ref.shape[1]
    ones_row = jnp.ones((1, n), jnp.float32)

    b1c = jax.lax.dot_general(b1_ref[...], ones_row, _OUTER,
                              preferred_element_type=jnp.float32)
    b2c = jax.lax.dot_general(b2_ref[...], ones_row, _OUTER,
                              preferred_element_type=jnp.float32)
    w1b = w1_ref[...].astype(jnp.bfloat16)
    w2b = w2_ref[...].astype(jnp.bfloat16)

    gs = range(_GPB)
    ms = [mask_ref[pl.ds(s * _GPB + g, 1), :] for g in gs]

    for g in gs:
        xp0t = jax.lax.dot_general(w1b, x_ref[g].astype(jnp.bfloat16), _NT,
                                   preferred_element_type=jnp.float32)
        xp0t_buf[g, 0:64, :] = xp0t.astype(jnp.bfloat16)
        xp0t_buf[g, 64:72, :] = jnp.ones((8, n), jnp.bfloat16)
        a_buf[g] = (adj_ref[g] > 0.98).astype(jnp.bfloat16)

    combs = [jax.lax.dot_general(xp0t_buf[g], a_buf[g], _NT,
                                 preferred_element_type=jnp.float32)
             for g in gs]
    for g in gs:
        comb = combs[g] + xp0t_buf[g].astype(jnp.float32)
        inv = 1.0 / comb[64:65, :]
        h1t_buf[g] = (jnp.maximum(comb[0:64, :] * inv + b1c, 0.0)
                      * ms[g]).astype(jnp.bfloat16)

    for g in gs:
        xp1t = jnp.dot(w2b, h1t_buf[g], preferred_element_type=jnp.float32)
        xp1t_buf[g, 0:32, :] = xp1t.astype(jnp.bfloat16)
        xp1t_buf[g, 32:40, :] = jnp.ones((8, n), jnp.bfloat16)

    combs2 = [jax.lax.dot_general(xp1t_buf[g], a_buf[g], _NT,
                                  preferred_element_type=jnp.float32)
              for g in gs]
    for g in gs:
        comb2 = combs2[g] + xp1t_buf[g].astype(jnp.float32)
        inv2 = 1.0 / comb2[32:33, :]
        h2 = jnp.maximum(comb2[0:32, :] * inv2 + b2c, 0.0) * ms[g]
        pooled = jnp.max(h2, axis=1, keepdims=True)
        outrow = jax.lax.dot_general(pooled, wfc_ref[...],
                                     (((0,), (1,)), ((), ())),
                                     preferred_element_type=jnp.float32) \
            + bfc_ref[...]
        out_ref[pl.ds(s * _GPB + g, 1), :] = outrow


def kernel(x, adj, mask, W1, b1, W2, b2, Wfc, bfc):
    B, N, F = x.shape
    H1 = W1.shape[0]
    H2 = W2.shape[0]
    OUT = Wfc.shape[0]

    return pl.pallas_call(
        _sage_kernel,
        grid=(B // _GPB,),
        in_specs=[
            pl.BlockSpec((_GPB, N, N), lambda s: (s, 0, 0)),
            pl.BlockSpec((_GPB, N, F), lambda s: (s, 0, 0)),
            pl.BlockSpec((B, N), lambda s: (0, 0)),
            pl.BlockSpec((H1, F), lambda s: (0, 0)),
            pl.BlockSpec((1, H1), lambda s: (0, 0)),
            pl.BlockSpec((H2, H1), lambda s: (0, 0)),
            pl.BlockSpec((1, H2), lambda s: (0, 0)),
            pl.BlockSpec((OUT, H2), lambda s: (0, 0)),
            pl.BlockSpec((1, OUT), lambda s: (0, 0)),
        ],
        out_specs=pl.BlockSpec((B, OUT), lambda s: (0, 0)),
        out_shape=jax.ShapeDtypeStruct((B, OUT), jnp.float32),
        scratch_shapes=[
            pltpu.VMEM((_GPB, N, N), jnp.bfloat16),
            pltpu.VMEM((_GPB, 72, N), jnp.bfloat16),
            pltpu.VMEM((_GPB, H1, N), jnp.bfloat16),
            pltpu.VMEM((_GPB, 40, N), jnp.bfloat16),
        ],
    )(adj, x, mask, W1, b1.reshape(1, H1), W2, b2.reshape(1, H2),
      Wfc, bfc.reshape(1, OUT))
